# Initial kernel scaffold; baseline (speedup 1.0000x reference)
import jax
import jax.numpy as jnp
from jax.experimental import pallas as pl


def kernel(h, edge_index, W1, Wl, W2, b2):
    raise NotImplementedError("write your pallas kernel here")


# speedup vs baseline: 3.4704x; 3.4704x over previous
import functools

import jax
import jax.numpy as jnp
from jax import lax
from jax.experimental import pallas as pl
from jax.experimental.pallas import tpu as pltpu
from jax.experimental.pallas import tpu_sc as plsc

N_NODES = 10000
N_EDGES = 160000
FEAT = 128
GRID = 4

ROW_BLOCK = 400
KDIM = FEAT * GRID
NCORES = 2
NSUB = 16
EDGES_PER_TILE = N_EDGES // (NCORES * NSUB)
CH = 128
FULL_CHUNKS = EDGES_PER_TILE // CH
TAIL = EDGES_PER_TILE - FULL_CHUNKS * CH
ROWS_PER_TILE = 624
ROWS_EXTRA = N_NODES - NSUB * ROWS_PER_TILE


def _kan_apply(x, wc_ref, ws_ref):
    xr = jnp.repeat(x, GRID, axis=1)
    kpat = (lax.broadcasted_iota(jnp.int32, xr.shape, 1) % GRID + 1
            ).astype(jnp.float32)
    arg = xr * kpat
    c = jnp.cos(arg).astype(jnp.bfloat16)
    s = jnp.sin(arg).astype(jnp.bfloat16)
    return (jnp.dot(c, wc_ref[...], preferred_element_type=jnp.float32) +
            jnp.dot(s, ws_ref[...], preferred_element_type=jnp.float32))


def _tc_kan_body(h_ref, w1c_ref, w1s_ref, wlc_ref, wls_ref, h1_ref, z_ref):
    h1 = _kan_apply(h_ref[...], w1c_ref, w1s_ref)
    h1_ref[...] = h1
    z_ref[...] = _kan_apply(h1, wlc_ref, wls_ref)


def _tc_kan(h, w1c, w1s, wlc, wls):
    nblk = N_NODES // ROW_BLOCK
    wspec = pl.BlockSpec((KDIM, FEAT), lambda i: (0, 0))
    return pl.pallas_call(
        _tc_kan_body,
        grid=(nblk,),
        in_specs=[
            pl.BlockSpec((ROW_BLOCK, FEAT), lambda i: (i, 0)),
            wspec, wspec, wspec, wspec,
        ],
        out_specs=[
            pl.BlockSpec((ROW_BLOCK, FEAT), lambda i: (i, 0)),
            pl.BlockSpec((ROW_BLOCK, FEAT), lambda i: (i, 0)),
        ],
        out_shape=[
            jax.ShapeDtypeStruct((N_NODES, FEAT), jnp.float32),
            jax.ShapeDtypeStruct((N_NODES, FEAT), jnp.float32),
        ],
    )(h, w1c, w1s, wlc, wls)


def _sc_scatter_body(z_hbm, src_hbm, dst_hbm, out_hbm,
                     src_v, dst_v, rows_v, src_t, dst_t, rows_t, m_sh, sem):
    c = lax.axis_index("c")
    s = lax.axis_index("s")

    def zrow(i, carry):
        for j in range(FEAT // 16):
            rows_v[i, pl.ds(j * 16, 16)] = jnp.zeros((16,), jnp.float32)
        return carry
    lax.fori_loop(0, CH, zrow, 0)
    base_r = pl.multiple_of(s * ROWS_PER_TILE, 8)
    off = 0
    while off < ROWS_PER_TILE:
        nn = min(CH, ROWS_PER_TILE - off)
        pltpu.sync_copy(rows_v.at[pl.ds(0, nn)],
                        m_sh.at[pl.ds(base_r + off, nn)])
        off += nn

    @pl.when(s == NSUB - 1)
    def _zero_extra():
        pltpu.sync_copy(rows_v.at[pl.ds(0, ROWS_EXTRA)],
                        m_sh.at[pl.ds(NSUB * ROWS_PER_TILE, ROWS_EXTRA)])
    plsc.subcore_barrier()

    ebase = (c * NSUB + s) * EDGES_PER_TILE

    def body(j, carry):
        base = pl.multiple_of(ebase + j * CH, 8)
        pltpu.sync_copy(src_hbm.at[pl.ds(base, CH)], src_v)
        pltpu.sync_copy(dst_hbm.at[pl.ds(base, CH)], dst_v)
        pltpu.async_copy(z_hbm.at[src_v], rows_v, sem).wait()
        pltpu.sync_copy(rows_v, m_sh.at[dst_v], add=True)
        return carry
    lax.fori_loop(0, FULL_CHUNKS, body, 0)

    if TAIL:
        tbase = pl.multiple_of(ebase + FULL_CHUNKS * CH, 8)
        pltpu.sync_copy(src_hbm.at[pl.ds(tbase, TAIL)], src_t)
        pltpu.sync_copy(dst_hbm.at[pl.ds(tbase, TAIL)], dst_t)
        pltpu.async_copy(z_hbm.at[src_t], rows_t, sem).wait()
        pltpu.sync_copy(rows_t, m_sh.at[dst_t], add=True)

    plsc.subcore_barrier()
    pltpu.sync_copy(m_sh.at[pl.ds(base_r, ROWS_PER_TILE)],
                    out_hbm.at[c].at[pl.ds(base_r, ROWS_PER_TILE)])

    @pl.when(s == NSUB - 1)
    def _write_extra():
        pltpu.sync_copy(m_sh.at[pl.ds(NSUB * ROWS_PER_TILE, ROWS_EXTRA)],
                        out_hbm.at[c].at[pl.ds(NSUB * ROWS_PER_TILE,
                                               ROWS_EXTRA)])


@functools.cache
def _sc_scatter():
    return pl.kernel(
        _sc_scatter_body,
        out_type=jax.ShapeDtypeStruct((NCORES, N_NODES, FEAT), jnp.float32),
        mesh=plsc.VectorSubcoreMesh(core_axis_name="c", subcore_axis_name="s"),
        scratch_types=[
            pltpu.VMEM((CH,), jnp.int32),
            pltpu.VMEM((CH,), jnp.int32),
            pltpu.VMEM((CH, FEAT), jnp.float32),
            pltpu.VMEM((TAIL,), jnp.int32),
            pltpu.VMEM((TAIL,), jnp.int32),
            pltpu.VMEM((TAIL, FEAT), jnp.float32),
            pltpu.VMEM_SHARED((N_NODES, FEAT), jnp.float32),
            pltpu.SemaphoreType.DMA,
        ],
    )


def _tc_final_body(m2_ref, h1_ref, w2c_ref, w2s_ref, b2_ref, out_ref, acc_ref):
    i = pl.program_id(0)

    @pl.when(i == 0)
    def _init():
        acc_ref[...] = jnp.zeros_like(acc_ref)

    t = m2_ref[0] + m2_ref[1] + h1_ref[...]
    h2 = jnp.where(t >= 0, t, jnp.float32(0.01) * t)
    acc_ref[...] += jnp.sum(h2, axis=0, keepdims=True)

    @pl.when(i == pl.num_programs(0) - 1)
    def _readout():
        y = acc_ref[...]
        cy = jnp.cos(y).astype(jnp.bfloat16).astype(jnp.float32)
        sy = jnp.sin(y).astype(jnp.bfloat16).astype(jnp.float32)
        w2c = w2c_ref[...].astype(jnp.bfloat16).astype(jnp.float32)
        w2s = w2s_ref[...].astype(jnp.bfloat16).astype(jnp.float32)
        logits = (jnp.sum(cy * w2c, axis=1, keepdims=True) +
                  jnp.sum(sy * w2s, axis=1, keepdims=True) + b2_ref[...])
        out_ref[...] = jax.nn.sigmoid(logits)


def _tc_final(m2, h1, w2c, w2s, b2):
    nblk = N_NODES // ROW_BLOCK
    return pl.pallas_call(
        _tc_final_body,
        grid=(nblk,),
        in_specs=[
            pl.BlockSpec((NCORES, ROW_BLOCK, FEAT), lambda i: (0, i, 0)),
            pl.BlockSpec((ROW_BLOCK, FEAT), lambda i: (i, 0)),
            pl.BlockSpec((1, FEAT), lambda i: (0, 0)),
            pl.BlockSpec((1, FEAT), lambda i: (0, 0)),
            pl.BlockSpec((1, 1), lambda i: (0, 0)),
        ],
        out_specs=pl.BlockSpec((1, 1), lambda i: (0, 0)),
        out_shape=jax.ShapeDtypeStruct((1, 1), jnp.float32),
        scratch_shapes=[pltpu.VMEM((1, FEAT), jnp.float32)],
    )(m2, h1, w2c, w2s, b2)


def kernel(h, edge_index, W1, Wl, W2, b2):
    src = edge_index[0].astype(jnp.int32)
    dst = edge_index[1].astype(jnp.int32)
    w1c = jnp.transpose(W1[0], (1, 2, 0)).reshape(KDIM, FEAT).astype(jnp.bfloat16)
    w1s = jnp.transpose(W1[1], (1, 2, 0)).reshape(KDIM, FEAT).astype(jnp.bfloat16)
    wlc = jnp.transpose(Wl[0], (1, 2, 0)).reshape(KDIM, FEAT).astype(jnp.bfloat16)
    wls = jnp.transpose(Wl[1], (1, 2, 0)).reshape(KDIM, FEAT).astype(jnp.bfloat16)
    w2c = W2[0, :, :, 0]
    w2s = W2[1, :, :, 0]

    h1, z = _tc_kan(h, w1c, w1s, wlc, wls)
    m2 = _sc_scatter()(z, src, dst)
    return _tc_final(m2, h1, w2c, w2s, b2)
